# Initial kernel scaffold; baseline (speedup 1.0000x reference)
import jax
import jax.numpy as jnp
from jax.experimental import pallas as pl


def kernel(x, y):
    raise NotImplementedError("write your pallas kernel here")


# speedup vs baseline: 45.0738x; 45.0738x over previous
import functools

import jax
import jax.numpy as jnp
from jax import lax
from jax.experimental import pallas as pl
from jax.experimental.pallas import tpu as pltpu
from jax.experimental.pallas import tpu_sc as plsc

_TOPK = 5
_ROWS = 128
_COLS = 32768
_RB = 8
_GRID = _ROWS // _RB
_NCHUNK = _COLS // 128
_NEG = float("-inf")


def _stage1_body(x_ref, e_ref, c_ref):
    neg = jnp.full((_RB, 128), _NEG, dtype=jnp.float32)
    acc_e = jnp.zeros((_RB, 128), dtype=jnp.float32)
    t1, t2, t3, t4, t5 = neg, neg, neg, neg, neg
    for c in range(_NCHUNK):
        v = x_ref[:, c * 128:(c + 1) * 128]
        acc_e = acc_e + jnp.exp(v)
        m = jnp.maximum(t1, v); v = jnp.minimum(t1, v); t1 = m
        m = jnp.maximum(t2, v); v = jnp.minimum(t2, v); t2 = m
        m = jnp.maximum(t3, v); v = jnp.minimum(t3, v); t3 = m
        m = jnp.maximum(t4, v); v = jnp.minimum(t4, v); t4 = m
        t5 = jnp.maximum(t5, v)
    e_ref[...] = jnp.sum(acc_e, axis=1, keepdims=True)
    c_ref[...] = jnp.concatenate([t1, t2, t3, t4, t5], axis=1)


def _stage2_body(c_ref, e_ref, s_ref, o_ref):
    cur = c_ref[...]
    e_tot = e_ref[...]
    s_y = s_ref[...]
    zero = jnp.zeros((_ROWS, 1), dtype=jnp.float32)
    tot4, tot5, cum, t4val = zero, zero, zero, zero
    for _ in range(_TOPK):
        v = jnp.max(cur, axis=1, keepdims=True)
        eq = cur == v
        cnt = jnp.sum(eq.astype(jnp.float32), axis=1, keepdims=True)
        cnt = jnp.where(v > _NEG, cnt, zero)
        take4 = jnp.minimum(cnt, jnp.maximum(4.0 - cum, 0.0))
        take5 = jnp.minimum(cnt, jnp.maximum(5.0 - cum, 0.0))
        ev = jnp.exp(v)
        tot4 = tot4 + ev * take4
        tot5 = tot5 + ev * take5
        new_cum = cum + cnt
        is_r4 = (cum < 4.0) & (new_cum >= 4.0)
        t4val = t4val + jnp.where(is_r4, v, zero)
        cum = new_cum
        cur = jnp.where(eq, _NEG, cur)
    d = jnp.where(s_y >= t4val, tot5 - jnp.exp(s_y), tot4)
    loss = jnp.log(e_tot - d) - s_y
    o_ref[...] = jnp.mean(loss).reshape(1, 1)


def _sc_gather(x_flat, flat_idx):
    mesh = plsc.VectorSubcoreMesh(core_axis_name="c", subcore_axis_name="s")
    per = _ROWS // 16

    @functools.partial(
        pl.kernel,
        out_type=jax.ShapeDtypeStruct((_ROWS,), jnp.float32),
        mesh=mesh,
        scratch_types=[
            pltpu.VMEM((per,), jnp.int32),
            pltpu.VMEM((per,), jnp.float32),
            pltpu.SemaphoreType.DMA,
        ],
    )
    def gather_kernel(x_hbm, i_hbm, o_hbm, idx_v, val_v, sem):
        cid = lax.axis_index("c")
        sid = lax.axis_index("s")

        @pl.when(cid == 0)
        def _():
            base = sid * per
            pltpu.sync_copy(i_hbm.at[pl.ds(base, per)], idx_v)
            pltpu.async_copy(x_hbm.at[idx_v], val_v, sem).wait()
            pltpu.sync_copy(val_v, o_hbm.at[pl.ds(base, per)])

    return gather_kernel(x_flat, flat_idx)


def kernel(x, y):
    flat_idx = (jnp.arange(_ROWS, dtype=jnp.int32) * _COLS
                + y.astype(jnp.int32))
    s_y = _sc_gather(x.reshape(-1), flat_idx)

    e_tot, cand = pl.pallas_call(
        _stage1_body,
        grid=(_GRID,),
        in_specs=[pl.BlockSpec((_RB, _COLS), lambda i: (i, 0))],
        out_specs=[
            pl.BlockSpec((_RB, 1), lambda i: (i, 0)),
            pl.BlockSpec((_RB, 5 * 128), lambda i: (i, 0)),
        ],
        out_shape=[
            jax.ShapeDtypeStruct((_ROWS, 1), jnp.float32),
            jax.ShapeDtypeStruct((_ROWS, 5 * 128), jnp.float32),
        ],
    )(x)

    out = pl.pallas_call(
        _stage2_body,
        grid=(1,),
        in_specs=[
            pl.BlockSpec((_ROWS, 5 * 128), lambda i: (0, 0)),
            pl.BlockSpec((_ROWS, 1), lambda i: (0, 0)),
            pl.BlockSpec((_ROWS, 1), lambda i: (0, 0)),
        ],
        out_specs=pl.BlockSpec((1, 1), lambda i: (0, 0)),
        out_shape=jax.ShapeDtypeStruct((1, 1), jnp.float32),
    )(cand, e_tot, s_y.reshape(_ROWS, 1))

    return out.reshape(())
